# baseline (speedup 1.0000x reference)
import jax
import jax.numpy as jnp
from jax.experimental import pallas as pl


def kernel(mem, idx, raw_msg, W1, b1, W2, b2, Wx, Wh, bx, bh):
    N = mem.shape[0]
    B = idx.shape[0]
    msg = jax.nn.relu(raw_msg @ W1 + b1) @ W2 + b2
    h = jnp.take(mem, idx, axis=0)
    gx = msg @ Wx + bx
    gh = h @ Wh + bh
    xr, xz, xn = jnp.split(gx, 3, axis=-1)
    hr, hz, hn = jnp.split(gh, 3, axis=-1)
    r = jax.nn.sigmoid(xr + hr)
    z = jax.nn.sigmoid(xz + hz)
    n = jnp.tanh(xn + r * hn)
    h_new = (1.0 - z) * n + z * h

    order = jnp.arange(B, dtype=jnp.int32)
    wmax = jnp.full((N,), -1, dtype=jnp.int32).at[idx].max(order)
    win = wmax[idx] == order
    safe_idx = jnp.where(win, idx, N)
    new_mem = mem.at[safe_idx].set(h_new, mode="drop")
    return new_mem


# speedup vs baseline: 1.1119x; 1.1119x over previous
import functools

import jax
import jax.numpy as jnp
from jax import lax
from jax.experimental import pallas as pl
from jax.experimental.pallas import tpu as pltpu

N = 100000
D = 256
B = 16384
RAW = 4 * D
HID = 2 * D
MSGP = 128

GRID = 32
BB = B // GRID
MB = 3200


def _tc_body(raw_ref, h_ref, mem_ref, W1_ref, b1_ref, W2_ref, b2_ref,
             Wx_ref, Wh_ref, bx_ref, bh_ref, out_mem_ref, h_new_ref):
    out_mem_ref[...] = mem_ref[...]

    f32 = jnp.float32
    x = jnp.maximum(
        jax.lax.dot(raw_ref[...], W1_ref[...], preferred_element_type=f32)
        + b1_ref[...], 0.0)
    msg = jax.lax.dot(x, W2_ref[...], preferred_element_type=f32) + b2_ref[...]
    gx = jax.lax.dot(msg, Wx_ref[...], preferred_element_type=f32) + bx_ref[...]
    h = h_ref[...]
    gh = jax.lax.dot(h, Wh_ref[...], preferred_element_type=f32) + bh_ref[...]
    xr, xz, xn = gx[:, :D], gx[:, D:2 * D], gx[:, 2 * D:]
    hr, hz, hn = gh[:, :D], gh[:, D:2 * D], gh[:, 2 * D:]
    r = jax.nn.sigmoid(xr + hr)
    z = jax.nn.sigmoid(xz + hz)
    n = jnp.tanh(xn + r * hn)
    h_new_ref[...] = (1.0 - z) * n + z * h


def _tc_call(raw_msg, h, mem, W1, b1, W2p, b2p, Wxp, Wh, bx, bh):
    full = lambda s: pl.BlockSpec(s, lambda b: (0, 0))
    return pl.pallas_call(
        _tc_body,
        grid=(GRID,),
        in_specs=[
            pl.BlockSpec((BB, RAW), lambda b: (b, 0)),
            pl.BlockSpec((BB, D), lambda b: (b, 0)),
            pl.BlockSpec((MB, D), lambda b: (b, 0)),
            full((RAW, HID)),
            full((1, HID)),
            full((HID, MSGP)),
            full((1, MSGP)),
            full((MSGP, 3 * D)),
            full((D, 3 * D)),
            full((1, 3 * D)),
            full((1, 3 * D)),
        ],
        out_specs=[
            pl.BlockSpec((MB, D), lambda b: (b, 0)),
            pl.BlockSpec((BB, D), lambda b: (b, 0)),
        ],
        out_shape=[
            jax.ShapeDtypeStruct((N, D), jnp.float32),
            jax.ShapeDtypeStruct((B, D), jnp.float32),
        ],
        compiler_params=pltpu.CompilerParams(
            dimension_semantics=("arbitrary",),
        ),
    )(raw_msg, h, mem, W1, b1, W2p, b2p, Wxp, Wh, bx, bh)


def kernel(mem, idx, raw_msg, W1, b1, W2, b2, Wx, Wh, bx, bh):
    MSG = W2.shape[1]
    W2p = jnp.zeros((HID, MSGP), jnp.float32).at[:, :MSG].set(W2)
    b2p = jnp.zeros((1, MSGP), jnp.float32).at[:, :MSG].set(b2)
    Wxp = jnp.zeros((MSGP, 3 * D), jnp.float32).at[:MSG].set(Wx)

    h = jnp.take(mem, idx, axis=0)

    out_mem, h_new = _tc_call(raw_msg, h, mem, W1, b1.reshape(1, -1), W2p,
                              b2p, Wxp, Wh, bx.reshape(1, -1),
                              bh.reshape(1, -1))

    order = jnp.arange(B, dtype=jnp.int32)
    wmax = jnp.full((N,), -1, dtype=jnp.int32).at[idx].max(order)
    win = wmax[idx] == order
    safe_idx = jnp.where(win, idx, N)
    return out_mem.at[safe_idx].set(h_new, mode="drop")
